# baseline (speedup 1.0000x reference)
import functools

import jax
import jax.numpy as jnp
from jax import lax
from jax.experimental import pallas as pl
from jax.experimental.pallas import tpu as pltpu
from jax.experimental.pallas import tpu_sc as plsc

BATCH = 16384
FIELDS = 26
DIM = 64
B_TOTAL = BATCH * FIELDS

NUM_CORES = 2
NUM_SUBCORES = 16
NW = NUM_CORES * NUM_SUBCORES
B_PER_W = B_TOTAL // NW

SUB = 128
GROUP = 512
GPG = GROUP // SUB
NGROUP = B_PER_W // GROUP
NBUF = 2


def _gather_descs(table_hbm, idx_v, rows_buf, sem, g):
    descs = []
    for j in range(GPG):
        off = pl.multiple_of(g * GROUP + j * SUB, SUB)
        descs.append(
            pltpu.make_async_copy(
                table_hbm.at[idx_v.at[pl.ds(off, SUB)]],
                rows_buf.at[pl.ds(j * SUB, SUB)],
                sem,
            )
        )
    return descs


def _body(table_hbm, idx_hbm, out_hbm, idx_v, rows0, rows1, sem0, sem1):
    wid = lax.axis_index("s") * NUM_CORES + lax.axis_index("c")
    base = pl.multiple_of(wid * B_PER_W, B_PER_W)
    pltpu.sync_copy(idx_hbm.at[pl.ds(base, B_PER_W)], idx_v)

    rows = (rows0, rows1)
    sems = (sem0, sem1)

    for b in range(NBUF):
        for d in _gather_descs(table_hbm, idx_v, rows[b], sems[b], b):
            d.start()

    def step(s, carry):
        g0 = s * NBUF
        for b in range(NBUF):
            g = g0 + b
            for d in _gather_descs(table_hbm, idx_v, rows[b], sems[b], g):
                d.wait()
            out_off = pl.multiple_of(base + g * GROUP, GROUP)
            pltpu.sync_copy(rows[b], out_hbm.at[pl.ds(out_off, GROUP)])
            nxt = g + NBUF

            @pl.when(nxt < NGROUP)
            def _():
                for d in _gather_descs(table_hbm, idx_v, rows[b], sems[b], nxt):
                    d.start()

        return carry

    lax.fori_loop(0, NGROUP // NBUF, step, 0)


@jax.jit
def _embedding_gather(table, idx_flat):
    mesh = plsc.VectorSubcoreMesh(core_axis_name="c", subcore_axis_name="s")
    k = functools.partial(
        pl.kernel,
        mesh=mesh,
        out_type=jax.ShapeDtypeStruct((B_TOTAL, DIM), jnp.float32),
        scratch_types=[
            pltpu.VMEM((B_PER_W,), jnp.int32),
            pltpu.VMEM((GROUP, DIM), jnp.float32),
            pltpu.VMEM((GROUP, DIM), jnp.float32),
            pltpu.SemaphoreType.DMA,
            pltpu.SemaphoreType.DMA,
        ],
        compiler_params=pltpu.CompilerParams(use_tc_tiling_on_sc=False),
    )(_body)
    return k(table, idx_flat)


def kernel(input_, table):
    idx_flat = input_.reshape(B_TOTAL)
    out = _embedding_gather(table, idx_flat)
    return out.reshape(BATCH, FIELDS, DIM)
